# baseline (speedup 1.0000x reference)
import jax
import jax.numpy as jnp
from jax import lax
from jax.experimental import pallas as pl
from jax.experimental.pallas import tpu as pltpu
from jax.experimental.pallas import tpu_sc as plsc

LANES = 16
NBIN = 65
NBUF = 4


def _build_sc_lookup(L, D):
  info = plsc.get_sparse_core_info()
  nc, ns = info.num_cores, info.num_subcores
  nw = nc * ns
  rpw = L // nw
  qlen = L // NBUF
  n_grp = qlen // LANES

  mesh = plsc.VectorSubcoreMesh(core_axis_name="c", subcore_axis_name="s")

  def body(idx_hbm, table_hbm, out_hbm, idx_v, table_v, slab, *sems):
    wid = lax.axis_index("s") * nc + lax.axis_index("c")
    base = wid * rpw
    pltpu.sync_copy(idx_hbm, idx_v.at[pl.ds(0, L)])
    pltpu.sync_copy(table_hbm, table_v)

    def row_step(r, carry):
      i = base + r
      s = idx_v[pl.ds(i, LANES)][0]
      for q in range(NBUF):
        slab_q = slab.at[jnp.int32(q)]

        @pl.when(r >= 1)
        def _drain(q=q, slab_q=slab_q):
          pltpu.make_async_copy(
              slab_q,
              out_hbm.at[jnp.int32(0), jnp.int32(0), pl.ds(0, qlen)],
              sems[q]).wait()

        def grp_step(g, carry2, q=q, slab_q=slab_q):
          jv = idx_v[pl.ds(q * qlen + g * LANES, LANES)]
          jb = jnp.clip(jv - s + 32, 0, NBIN - 1)
          jb_d = jb * D
          grow = g * LANES
          nt = D // LANES
          for k0 in range(0, LANES, 8):
            addrs = [jb_d[k0 + m] for m in range(8)]
            vals = [table_v[pl.ds(addrs[m] + t * LANES, LANES)]
                    for m in range(8) for t in range(nt)]
            for m in range(8):
              for t in range(nt):
                slab_q[grow + k0 + m, pl.ds(t * LANES, LANES)] = (
                    vals[m * nt + t])
          return carry2

        lax.fori_loop(jnp.int32(0), jnp.int32(n_grp), grp_step, jnp.int32(0))
        pltpu.async_copy(
            slab_q,
            out_hbm.at[jnp.int32(0), i, pl.ds(q * qlen, qlen)],
            sems[q])
      return carry

    lax.fori_loop(jnp.int32(0), jnp.int32(rpw), row_step, jnp.int32(0))
    for q in range(NBUF):
      pltpu.make_async_copy(
          slab.at[jnp.int32(q)],
          out_hbm.at[jnp.int32(0), jnp.int32(0), pl.ds(0, qlen)],
          sems[q]).wait()

  return pl.kernel(
      body,
      mesh=mesh,
      compiler_params=pltpu.CompilerParams(
          use_tc_tiling_on_sc=False, needs_layout_passes=False),
      out_type=jax.ShapeDtypeStruct((1, L, L, D), jnp.float32),
      scratch_types=[
          pltpu.VMEM((L + LANES,), jnp.int32),
          pltpu.VMEM((NBIN * D,), jnp.float32),
          pltpu.VMEM((NBUF, qlen, D), jnp.float32),
      ] + [pltpu.SemaphoreType.DMA] * NBUF,
  )


def kernel(idx, stride, emb_weight):
  B, L = idx.shape
  D = emb_weight.shape[-1]
  idx32 = idx.reshape(L).astype(jnp.int32)
  table_flat = emb_weight.astype(jnp.float32).reshape(NBIN * D)
  return _build_sc_lookup(L, D)(idx32, table_flat)
